# Initial kernel scaffold; baseline (speedup 1.0000x reference)
import jax
import jax.numpy as jnp
from jax.experimental import pallas as pl


def kernel(scores):
    raise NotImplementedError("write your pallas kernel here")


# speedup vs baseline: 4.4667x; 4.4667x over previous
import functools
import math

import jax
import jax.numpy as jnp
from jax import lax
from jax.experimental import pallas as pl
from jax.experimental.pallas import tpu as pltpu
from jax.experimental.pallas import tpu_sc as plsc

_K = 8
_N = 32
_LANES = 16
_NC = 2
_NS = 16
_NW = _NC * _NS
_GROUPS_PER_W = 196
_G = _NW * _GROUPS_PER_W
_RPAD = _G * _LANES


def _sc_body(theta_hbm, u_hbm, marg_hbm, samp_hbm,
             theta_v, u_v, w_v, btab, marg_v, samp_v):
    wid = lax.axis_index("s") * _NC + lax.axis_index("c")
    lane = lax.iota(jnp.int32, _LANES)

    def group(gi, _):
        g = wid * _GROUPS_PER_W + gi
        pltpu.sync_copy(theta_hbm.at[g], theta_v)
        pltpu.sync_copy(u_hbm.at[g], u_v)

        for i in range(_N):
            w_v[i] = jnp.exp(theta_v[i])

        ones = jnp.full((_LANES,), 1.0, jnp.float32)
        zero = jnp.zeros((_LANES,), jnp.float32)
        b = [ones] + [zero] * _K
        for j in range(_K + 1):
            btab[_N * (_K + 1) + j] = b[j]
        for i in range(_N - 1, -1, -1):
            wi = w_v[i]
            hi = min(_K, _N - i)
            for k in range(hi, 0, -1):
                b[k] = b[k] + b[k - 1] * wi
            for j in range(_K + 1):
                btab[i * (_K + 1) + j] = b[j]

        f = [ones] + [zero] * _K
        for i in range(_N):
            wi = w_v[i]
            jmax = min(i, _K - 1)
            num = f[0] * btab[(i + 1) * (_K + 1) + (_K - 1)]
            for j in range(1, jmax + 1):
                num = num + f[j] * btab[(i + 1) * (_K + 1) + (_K - 1 - j)]
            marg_v[i] = wi * num
            hi = min(_K, i + 1)
            for k in range(hi, 0, -1):
                f[k] = f[k] + f[k - 1] * wi
        inv = 1.0 / f[_K]
        for i in range(_N):
            marg_v[i] = marg_v[i] * inv

        r = jnp.full((_LANES,), _K, jnp.int32)
        for i in range(_N):
            rm1 = jnp.clip(r - 1, 0, _K)
            rc = jnp.clip(r, 0, _K)
            g1 = plsc.load_gather(btab, [(i + 1) * (_K + 1) + rm1, lane])
            g2 = plsc.load_gather(btab, [i * (_K + 1) + rc, lane])
            wi = w_v[i]
            p = jnp.clip((wi * g1) / g2, 0.0, 1.0)
            p = jnp.where(g2 == 0.0, jnp.minimum(wi, 1.0), p)
            p = jnp.where(r > 0, p, 0.0)
            take = u_v[i] < p
            samp_v[i] = jnp.where(take, 1.0, 0.0)
            r = r - jnp.where(take, 1, 0).astype(jnp.int32)

        pltpu.sync_copy(marg_v, marg_hbm.at[g])
        pltpu.sync_copy(samp_v, samp_hbm.at[g])
        return ()

    lax.fori_loop(0, _GROUPS_PER_W, group, (), unroll=False)


@jax.jit
def kernel(scores):
    nnodes, choices, ensemble = scores.shape
    assert choices == _N and 2 ** int(math.log2(choices)) == choices
    rows = nnodes * ensemble
    theta = jnp.transpose(scores, (0, 2, 1)).reshape(rows, choices)

    u = jax.random.uniform(jax.random.key(42), (_N, 1, rows), dtype=theta.dtype)
    u2 = u[:, 0, :]

    pad = _RPAD - rows
    theta_p = jnp.pad(theta, ((0, pad), (0, 0)))
    u_p = jnp.pad(u2, ((0, 0), (0, pad)), constant_values=0.5)
    theta_b = theta_p.reshape(_G, _LANES, _N).transpose(0, 2, 1)
    u_b = u_p.reshape(_N, _G, _LANES).transpose(1, 0, 2)

    mesh = plsc.VectorSubcoreMesh(core_axis_name="c", subcore_axis_name="s",
                                  num_cores=_NC, num_subcores=_NS)
    marg_b, samp_b = pl.kernel(
        _sc_body,
        out_type=[
            jax.ShapeDtypeStruct((_G, _N, _LANES), jnp.float32),
            jax.ShapeDtypeStruct((_G, _N, _LANES), jnp.float32),
        ],
        mesh=mesh,
        compiler_params=pltpu.CompilerParams(needs_layout_passes=False),
        scratch_types=[
            pltpu.VMEM((_N, _LANES), jnp.float32),
            pltpu.VMEM((_N, _LANES), jnp.float32),
            pltpu.VMEM((_N, _LANES), jnp.float32),
            pltpu.VMEM(((_N + 1) * (_K + 1), _LANES), jnp.float32),
            pltpu.VMEM((_N, _LANES), jnp.float32),
            pltpu.VMEM((_N, _LANES), jnp.float32),
        ],
    )(theta_b, u_b)

    marg_flat = marg_b.transpose(0, 2, 1).reshape(_RPAD, _N)[:rows]
    samp_flat = samp_b.transpose(0, 2, 1).reshape(_RPAD, _N)[:rows]
    marginals = jnp.transpose(marg_flat.reshape(nnodes, ensemble, choices), (0, 2, 1))
    samples = jnp.transpose(samp_flat.reshape(nnodes, ensemble, choices), (0, 2, 1))[None]
    return samples, marginals


# speedup vs baseline: 4.6449x; 1.0399x over previous
import functools
import math

import jax
import jax.numpy as jnp
from jax import lax
from jax.experimental import pallas as pl
from jax.experimental.pallas import tpu as pltpu
from jax.experimental.pallas import tpu_sc as plsc

_K = 8
_N = 32
_LANES = 16
_NC = 2
_NS = 16
_NW = _NC * _NS
_GROUPS_PER_W = 196
_G = _NW * _GROUPS_PER_W
_RPAD = _G * _LANES


def _sc_body(theta_hbm, u_hbm, marg_hbm, samp_hbm,
             theta_v, u_v, w_v, btab, marg_v, samp_v):
    wid = lax.axis_index("s") * _NC + lax.axis_index("c")
    lane = lax.iota(jnp.int32, _LANES)
    ones = jnp.full((_LANES,), 1.0, jnp.float32)
    zero = jnp.zeros((_LANES,), jnp.float32)

    for i in range(_N + 1):
        btab[i * (_K + 1)] = ones
        for j in range(min(_K, _N - i) + 1, _K + 1):
            btab[i * (_K + 1) + j] = zero

    def group(gi, _):
        g = wid * _GROUPS_PER_W + gi
        pltpu.sync_copy(theta_hbm.at[g], theta_v)
        pltpu.sync_copy(u_hbm.at[g], u_v)

        for i in range(_N):
            w_v[i] = jnp.exp(theta_v[i])

        b = [ones] + [zero] * _K
        for i in range(_N - 1, -1, -1):
            wi = w_v[i]
            hi = min(_K, _N - i)
            for k in range(hi, 0, -1):
                b[k] = b[k] + b[k - 1] * wi
            for j in range(1, hi + 1):
                btab[i * (_K + 1) + j] = b[j]

        f = [ones] + [zero] * _K
        for i in range(_N):
            wi = w_v[i]
            jlo = max(0, i - (_N - _K))
            jhi = min(i, _K - 1)
            num = f[jlo] * btab[(i + 1) * (_K + 1) + (_K - 1 - jlo)]
            for j in range(jlo + 1, jhi + 1):
                num = num + f[j] * btab[(i + 1) * (_K + 1) + (_K - 1 - j)]
            marg_v[i] = wi * num
            hi = min(_K, i + 1)
            for k in range(hi, 0, -1):
                f[k] = f[k] + f[k - 1] * wi
        inv = 1.0 / f[_K]
        for i in range(_N):
            marg_v[i] = marg_v[i] * inv

        r = jnp.full((_LANES,), _K, jnp.int32)
        for i in range(_N):
            rm1 = jnp.maximum(r - 1, 0)
            g1 = plsc.load_gather(btab, [(i + 1) * (_K + 1) + rm1, lane])
            g2 = plsc.load_gather(btab, [i * (_K + 1) + r, lane])
            wi = w_v[i]
            ui = u_v[i]
            take_main = ui * g2 < wi * g1
            take_edge = ui < jnp.minimum(wi, 1.0)
            take = jnp.where(g2 == 0.0, take_edge, take_main) & (r > 0)
            samp_v[i] = jnp.where(take, 1.0, 0.0)
            r = r - jnp.where(take, 1, 0).astype(jnp.int32)

        pltpu.sync_copy(marg_v, marg_hbm.at[g])
        pltpu.sync_copy(samp_v, samp_hbm.at[g])
        return ()

    lax.fori_loop(0, _GROUPS_PER_W, group, (), unroll=False)


@jax.jit
def kernel(scores):
    nnodes, choices, ensemble = scores.shape
    assert choices == _N and 2 ** int(math.log2(choices)) == choices
    rows = nnodes * ensemble
    theta = jnp.transpose(scores, (0, 2, 1)).reshape(rows, choices)

    u = jax.random.uniform(jax.random.key(42), (_N, 1, rows), dtype=theta.dtype)
    u2 = u[:, 0, :]

    pad = _RPAD - rows
    theta_p = jnp.pad(theta, ((0, pad), (0, 0)))
    u_p = jnp.pad(u2, ((0, 0), (0, pad)), constant_values=0.5)
    theta_b = theta_p.reshape(_G, _LANES, _N).transpose(0, 2, 1)
    u_b = u_p.reshape(_N, _G, _LANES).transpose(1, 0, 2)

    mesh = plsc.VectorSubcoreMesh(core_axis_name="c", subcore_axis_name="s",
                                  num_cores=_NC, num_subcores=_NS)
    marg_b, samp_b = pl.kernel(
        _sc_body,
        out_type=[
            jax.ShapeDtypeStruct((_G, _N, _LANES), jnp.float32),
            jax.ShapeDtypeStruct((_G, _N, _LANES), jnp.float32),
        ],
        mesh=mesh,
        compiler_params=pltpu.CompilerParams(needs_layout_passes=False),
        scratch_types=[
            pltpu.VMEM((_N, _LANES), jnp.float32),
            pltpu.VMEM((_N, _LANES), jnp.float32),
            pltpu.VMEM((_N, _LANES), jnp.float32),
            pltpu.VMEM(((_N + 1) * (_K + 1), _LANES), jnp.float32),
            pltpu.VMEM((_N, _LANES), jnp.float32),
            pltpu.VMEM((_N, _LANES), jnp.float32),
        ],
    )(theta_b, u_b)

    marg_flat = marg_b.transpose(0, 2, 1).reshape(_RPAD, _N)[:rows]
    samp_flat = samp_b.transpose(0, 2, 1).reshape(_RPAD, _N)[:rows]
    marginals = jnp.transpose(marg_flat.reshape(nnodes, ensemble, choices), (0, 2, 1))
    samples = jnp.transpose(samp_flat.reshape(nnodes, ensemble, choices), (0, 2, 1))[None]
    return samples, marginals
